# baseline (speedup 1.0000x reference)
from functools import partial

import jax
import jax.numpy as jnp
from jax.experimental import pallas as pl
from jax.experimental.pallas import tpu as pltpu


def _interleave_phases(ys, H, W):
    Cout = ys[0].shape[0]
    src = jnp.concatenate(
        [y.reshape(Cout, H, W) for y in ys], axis=-1)
    g = jax.lax.broadcasted_iota(jnp.int32, (Cout, H, 4 * W), 2)
    half = g // (2 * W)
    gg = g % (2 * W)
    p = 2 * (1 - half) + (1 - gg % 2)
    return jnp.take_along_axis(src, p * W + gg // 2, axis=-1)


def _conv_stats_kernel(x_ref, w_ref, b_ref, o_ref, sum_ref, ssq_ref, *, H, W):
    xb = x_ref[0]
    lane = jax.lax.broadcasted_iota(jnp.int32, xb.shape, 1)
    col = lane % W

    taps = {}
    for dh in (-1, 0, 1):
        for dw in (-1, 0, 1):
            k = dh * W + dw
            t = xb if k == 0 else jnp.roll(xb, -k, axis=1)
            masks = []
            if dh == 1:
                masks.append(lane < (H - 1) * W)
            elif dh == -1:
                masks.append(lane >= W)
            if dw == 1:
                masks.append(col < (W - 1))
            elif dw == -1:
                masks.append(col >= 1)
            if masks:
                m = masks[0]
                for mm in masks[1:]:
                    m = jnp.logical_and(m, mm)
                t = jnp.where(m, t, 0.0)
            taps[(dh, dw)] = t

    b = b_ref[...]
    ssum = jnp.zeros_like(b)
    ssq = jnp.zeros_like(b)
    ys = []
    p = 0
    for rh in (0, 1):
        ch = 1 - rh
        for rw in (0, 1):
            cw = 1 - rw
            acc = None
            for mh in (0, 1):
                for mw in (0, 1):
                    d = jnp.dot(w_ref[p, mh * 2 + mw],
                                taps[(ch - mh, cw - mw)],
                                preferred_element_type=jnp.float32)
                    acc = d if acc is None else acc + d
            y = acc + b
            ys.append(y)
            ssum = ssum + jnp.sum(y, axis=1, keepdims=True)
            ssq = ssq + jnp.sum(y * y, axis=1, keepdims=True)
            p += 1
    o_ref[0] = _interleave_phases(ys, H, W)
    sum_ref[0] = ssum
    ssq_ref[0] = ssq


def _bn_relu_kernel(c_ref, sum_ref, ssq_ref, g_ref, be_ref, o_ref, *, Mtot,
                    eps):
    mean = jnp.sum(sum_ref[...], axis=0) / Mtot
    var = jnp.maximum(jnp.sum(ssq_ref[...], axis=0) / Mtot - mean * mean, 0.0)
    inv = jax.lax.rsqrt(var + eps)
    scale = (g_ref[...] * inv)[:, :, None]
    shift = (be_ref[...] - mean * g_ref[...] * inv)[:, :, None]
    o_ref[0] = jnp.maximum(c_ref[0] * scale + shift, 0.0)


def _upsample(x, w_t, bias, gamma, beta, *, eps=1e-5):
    N, Cin, H, W = map(int, x.shape)
    _, Cout, K, _ = map(int, w_t.shape)
    assert K == 4
    HW = H * W
    P = 4

    xf = x.reshape(N, Cin, HW)

    wms = (w_t.transpose(2, 3, 1, 0)
           .reshape(2, 2, 2, 2, Cout, Cin)
           .transpose(1, 3, 0, 2, 4, 5)
           .reshape(P, 4, Cout, Cin).astype(jnp.float32))
    b2 = bias.reshape(Cout, 1).astype(jnp.float32)

    craw, sums, ssq = pl.pallas_call(
        partial(_conv_stats_kernel, H=H, W=W),
        out_shape=(
            jax.ShapeDtypeStruct((N, Cout, H, 4 * W), jnp.float32),
            jax.ShapeDtypeStruct((N, Cout, 1), jnp.float32),
            jax.ShapeDtypeStruct((N, Cout, 1), jnp.float32),
        ),
        grid=(N,),
        in_specs=[
            pl.BlockSpec((1, Cin, HW), lambda n: (n, 0, 0)),
            pl.BlockSpec((P, 4, Cout, Cin), lambda n: (0, 0, 0, 0)),
            pl.BlockSpec((Cout, 1), lambda n: (0, 0)),
        ],
        out_specs=(
            pl.BlockSpec((1, Cout, H, 4 * W), lambda n: (n, 0, 0, 0)),
            pl.BlockSpec((1, Cout, 1), lambda n: (n, 0, 0)),
            pl.BlockSpec((1, Cout, 1), lambda n: (n, 0, 0)),
        ),
        compiler_params=pltpu.CompilerParams(
            dimension_semantics=("parallel",)),
    )(xf, wms, b2)

    g2 = gamma.reshape(Cout, 1).astype(jnp.float32)
    be2 = beta.reshape(Cout, 1).astype(jnp.float32)

    out = pl.pallas_call(
        partial(_bn_relu_kernel, Mtot=float(N * P * HW), eps=eps),
        out_shape=jax.ShapeDtypeStruct((N, Cout, H, 4 * W), jnp.float32),
        grid=(N,),
        in_specs=[
            pl.BlockSpec((1, Cout, H, 4 * W), lambda n: (n, 0, 0, 0)),
            pl.BlockSpec((N, Cout, 1), lambda n: (0, 0, 0)),
            pl.BlockSpec((N, Cout, 1), lambda n: (0, 0, 0)),
            pl.BlockSpec((Cout, 1), lambda n: (0, 0)),
            pl.BlockSpec((Cout, 1), lambda n: (0, 0)),
        ],
        out_specs=pl.BlockSpec((1, Cout, H, 4 * W), lambda n: (n, 0, 0, 0)),
        compiler_params=pltpu.CompilerParams(
            dimension_semantics=("parallel",)),
    )(craw, sums, ssq, g2, be2)

    return out.reshape(N, Cout, 2 * H, 2 * W)


def kernel(x, w_t, bias, gamma, beta):
    return _upsample(x, w_t, bias, gamma, beta)


# speedup vs baseline: 1.1364x; 1.1364x over previous
from functools import partial

import jax
import jax.numpy as jnp
from jax.experimental import pallas as pl
from jax.experimental.pallas import tpu as pltpu


def _conv_stats_kernel(x_ref, w_ref, b_ref, o_ref, sum_ref, ssq_ref, *, H, W):
    xb = x_ref[0]
    lane = jax.lax.broadcasted_iota(jnp.int32, xb.shape, 1)
    col = lane % W

    taps = {}
    for dh in (-1, 0, 1):
        for dw in (-1, 0, 1):
            k = dh * W + dw
            t = xb if k == 0 else jnp.roll(xb, -k, axis=1)
            masks = []
            if dh == 1:
                masks.append(lane < (H - 1) * W)
            elif dh == -1:
                masks.append(lane >= W)
            if dw == 1:
                masks.append(col < (W - 1))
            elif dw == -1:
                masks.append(col >= 1)
            if masks:
                m = masks[0]
                for mm in masks[1:]:
                    m = jnp.logical_and(m, mm)
                t = jnp.where(m, t, 0.0)
            taps[(dh, dw)] = t

    b = b_ref[...]
    ssum = jnp.zeros_like(b)
    ssq = jnp.zeros_like(b)
    p = 0
    for rh in (0, 1):
        ch = 1 - rh
        for rw in (0, 1):
            cw = 1 - rw
            acc = None
            for mh in (0, 1):
                for mw in (0, 1):
                    d = jnp.dot(w_ref[p, mh * 2 + mw],
                                taps[(ch - mh, cw - mw)],
                                preferred_element_type=jnp.float32)
                    acc = d if acc is None else acc + d
            y = acc + b
            o_ref[0, p] = y
            ssum = ssum + jnp.sum(y, axis=1, keepdims=True)
            ssq = ssq + jnp.sum(y * y, axis=1, keepdims=True)
            p += 1
    sum_ref[0] = ssum
    ssq_ref[0] = ssq


def _bn_relu_interleave_kernel(c_ref, sc_ref, sh_ref, o_ref, *, H, W):
    y = jnp.maximum(c_ref[0] * sc_ref[...] + sh_ref[...], 0.0)
    Cout = y.shape[1]
    src = jnp.concatenate(
        [y[p].reshape(Cout, H, W) for p in range(4)], axis=-1)
    g = jax.lax.broadcasted_iota(jnp.int32, (Cout, H, 4 * W), 2)
    half = g // (2 * W)
    gg = g % (2 * W)
    p = 2 * (1 - half) + (1 - gg % 2)
    idx = p * W + gg // 2
    o_ref[0] = jnp.take_along_axis(src, idx, axis=-1)


def _upsample(x, w_t, bias, gamma, beta, *, eps=1e-5):
    N, Cin, H, W = map(int, x.shape)
    _, Cout, K, _ = map(int, w_t.shape)
    assert K == 4
    HW = H * W
    P = 4

    xf = x.reshape(N, Cin, HW)

    wms = (w_t.transpose(2, 3, 1, 0)
           .reshape(2, 2, 2, 2, Cout, Cin)
           .transpose(1, 3, 0, 2, 4, 5)
           .reshape(P, 4, Cout, Cin).astype(jnp.float32))
    b2 = bias.reshape(Cout, 1).astype(jnp.float32)

    conv, sums, ssq = pl.pallas_call(
        partial(_conv_stats_kernel, H=H, W=W),
        out_shape=(
            jax.ShapeDtypeStruct((N, P, Cout, HW), jnp.float32),
            jax.ShapeDtypeStruct((N, Cout, 1), jnp.float32),
            jax.ShapeDtypeStruct((N, Cout, 1), jnp.float32),
        ),
        grid=(N,),
        in_specs=[
            pl.BlockSpec((1, Cin, HW), lambda n: (n, 0, 0)),
            pl.BlockSpec((P, 4, Cout, Cin), lambda n: (0, 0, 0, 0)),
            pl.BlockSpec((Cout, 1), lambda n: (0, 0)),
        ],
        out_specs=(
            pl.BlockSpec((1, P, Cout, HW), lambda n: (n, 0, 0, 0)),
            pl.BlockSpec((1, Cout, 1), lambda n: (n, 0, 0)),
            pl.BlockSpec((1, Cout, 1), lambda n: (n, 0, 0)),
        ),
        compiler_params=pltpu.CompilerParams(
            dimension_semantics=("parallel",)),
    )(xf, wms, b2)

    Mtot = float(N * P * HW)
    mean = jnp.sum(sums, axis=0) / Mtot
    var = jnp.maximum(jnp.sum(ssq, axis=0) / Mtot - mean * mean, 0.0)
    inv = jax.lax.rsqrt(var + eps)
    scale = gamma.reshape(Cout, 1).astype(jnp.float32) * inv
    shift = beta.reshape(Cout, 1).astype(jnp.float32) - mean * scale

    out = pl.pallas_call(
        partial(_bn_relu_interleave_kernel, H=H, W=W),
        out_shape=jax.ShapeDtypeStruct((N, Cout, H, 4 * W), jnp.float32),
        grid=(N,),
        in_specs=[
            pl.BlockSpec((1, P, Cout, HW), lambda n: (n, 0, 0, 0)),
            pl.BlockSpec((Cout, 1), lambda n: (0, 0)),
            pl.BlockSpec((Cout, 1), lambda n: (0, 0)),
        ],
        out_specs=pl.BlockSpec((1, Cout, H, 4 * W), lambda n: (n, 0, 0, 0)),
        compiler_params=pltpu.CompilerParams(
            dimension_semantics=("parallel",)),
    )(conv, scale, shift)

    return out.reshape(N, Cout, 2 * H, 2 * W)


def kernel(x, w_t, bias, gamma, beta):
    return _upsample(x, w_t, bias, gamma, beta)


# speedup vs baseline: 1.1458x; 1.0083x over previous
from functools import partial

import jax
import jax.numpy as jnp
from jax.experimental import pallas as pl
from jax.experimental.pallas import tpu as pltpu


def _conv_stats_kernel(x_ref, w_ref, b_ref, o_ref, sum_ref, ssq_ref, *, H, W):
    HW = H * W
    xb = jnp.concatenate([x_ref[0, 0], x_ref[0, 1]], axis=-1)
    lane = jax.lax.broadcasted_iota(jnp.int32, xb.shape, 1)
    li = lane % HW
    col = lane % W

    taps = {}
    for dh in (-1, 0, 1):
        for dw in (-1, 0, 1):
            k = dh * W + dw
            t = xb if k == 0 else jnp.roll(xb, -k, axis=1)
            masks = []
            if dh == 1:
                masks.append(li < (H - 1) * W)
            elif dh == -1:
                masks.append(li >= W)
            if dw == 1:
                masks.append(col < (W - 1))
            elif dw == -1:
                masks.append(col >= 1)
            if masks:
                m = masks[0]
                for mm in masks[1:]:
                    m = jnp.logical_and(m, mm)
                t = jnp.where(m, t, 0.0)
            taps[(dh, dw)] = t

    b = b_ref[...]
    ssum = jnp.zeros_like(b)
    ssq = jnp.zeros_like(b)
    p = 0
    for rh in (0, 1):
        ch = 1 - rh
        for rw in (0, 1):
            cw = 1 - rw
            acc = None
            for mh in (0, 1):
                for mw in (0, 1):
                    d = jnp.dot(w_ref[p, mh * 2 + mw],
                                taps[(ch - mh, cw - mw)],
                                preferred_element_type=jnp.float32)
                    acc = d if acc is None else acc + d
            y = acc + b
            o_ref[0, p] = y
            ssum = ssum + jnp.sum(y, axis=1, keepdims=True)
            ssq = ssq + jnp.sum(y * y, axis=1, keepdims=True)
            p += 1
    sum_ref[0] = ssum
    ssq_ref[0] = ssq


def _bn_relu_interleave_kernel(c_ref, sc_ref, sh_ref, o_ref, *, H, W):
    HW = H * W
    y = jnp.maximum(c_ref[0] * sc_ref[...] + sh_ref[...], 0.0)
    Cout = y.shape[1]
    g = jax.lax.broadcasted_iota(jnp.int32, (Cout, H, 4 * W), 2)
    half = g // (2 * W)
    gg = g % (2 * W)
    idx = (2 * (1 - half) + (1 - gg % 2)) * W + gg // 2
    for ii in (0, 1):
        yi = y[:, :, ii * HW:(ii + 1) * HW]
        src = jnp.concatenate(
            [yi[p].reshape(Cout, H, W) for p in range(4)], axis=-1)
        o_ref[ii] = jnp.take_along_axis(src, idx, axis=-1)


def _upsample(x, w_t, bias, gamma, beta, *, eps=1e-5):
    N, Cin, H, W = map(int, x.shape)
    _, Cout, K, _ = map(int, w_t.shape)
    assert K == 4 and N % 2 == 0
    HW = H * W
    P = 4
    NP = N // 2

    xp = x.reshape(NP, 2, Cin, HW)

    wms = (w_t.transpose(2, 3, 1, 0)
           .reshape(2, 2, 2, 2, Cout, Cin)
           .transpose(1, 3, 0, 2, 4, 5)
           .reshape(P, 4, Cout, Cin).astype(jnp.float32))
    b2 = bias.reshape(Cout, 1).astype(jnp.float32)

    conv, sums, ssq = pl.pallas_call(
        partial(_conv_stats_kernel, H=H, W=W),
        out_shape=(
            jax.ShapeDtypeStruct((NP, P, Cout, 2 * HW), jnp.float32),
            jax.ShapeDtypeStruct((NP, Cout, 1), jnp.float32),
            jax.ShapeDtypeStruct((NP, Cout, 1), jnp.float32),
        ),
        grid=(NP,),
        in_specs=[
            pl.BlockSpec((1, 2, Cin, HW), lambda n: (n, 0, 0, 0)),
            pl.BlockSpec((P, 4, Cout, Cin), lambda n: (0, 0, 0, 0)),
            pl.BlockSpec((Cout, 1), lambda n: (0, 0)),
        ],
        out_specs=(
            pl.BlockSpec((1, P, Cout, 2 * HW), lambda n: (n, 0, 0, 0)),
            pl.BlockSpec((1, Cout, 1), lambda n: (n, 0, 0)),
            pl.BlockSpec((1, Cout, 1), lambda n: (n, 0, 0)),
        ),
        compiler_params=pltpu.CompilerParams(
            dimension_semantics=("parallel",)),
    )(xp, wms, b2)

    Mtot = float(N * P * HW)
    mean = jnp.sum(sums, axis=0) / Mtot
    var = jnp.maximum(jnp.sum(ssq, axis=0) / Mtot - mean * mean, 0.0)
    inv = jax.lax.rsqrt(var + eps)
    scale = gamma.reshape(Cout, 1).astype(jnp.float32) * inv
    shift = beta.reshape(Cout, 1).astype(jnp.float32) - mean * scale

    out = pl.pallas_call(
        partial(_bn_relu_interleave_kernel, H=H, W=W),
        out_shape=jax.ShapeDtypeStruct((N, Cout, H, 4 * W), jnp.float32),
        grid=(NP,),
        in_specs=[
            pl.BlockSpec((1, P, Cout, 2 * HW), lambda n: (n, 0, 0, 0)),
            pl.BlockSpec((Cout, 1), lambda n: (0, 0)),
            pl.BlockSpec((Cout, 1), lambda n: (0, 0)),
        ],
        out_specs=pl.BlockSpec((2, Cout, H, 4 * W), lambda n: (n, 0, 0, 0)),
        compiler_params=pltpu.CompilerParams(
            dimension_semantics=("parallel",)),
    )(conv, scale, shift)

    return out.reshape(N, Cout, 2 * H, 2 * W)


def kernel(x, w_t, bias, gamma, beta):
    return _upsample(x, w_t, bias, gamma, beta)


# speedup vs baseline: 1.1481x; 1.0020x over previous
from functools import partial

import jax
import jax.numpy as jnp
from jax.experimental import pallas as pl
from jax.experimental.pallas import tpu as pltpu


def _conv_stats_kernel(x_ref, w_ref, b_ref, o_ref, sum_ref, ssq_ref, *, H, W):
    HW = H * W
    xb = jnp.concatenate([x_ref[0, 0], x_ref[0, 1]], axis=-1)
    lane = jax.lax.broadcasted_iota(jnp.int32, xb.shape, 1)
    li = lane % HW
    col = lane % W

    taps = {}
    for dh in (-1, 0, 1):
        for dw in (-1, 0, 1):
            k = dh * W + dw
            t = xb if k == 0 else jnp.roll(xb, -k, axis=1)
            masks = []
            if dh == 1:
                masks.append(li < (H - 1) * W)
            elif dh == -1:
                masks.append(li >= W)
            if dw == 1:
                masks.append(col < (W - 1))
            elif dw == -1:
                masks.append(col >= 1)
            if masks:
                m = masks[0]
                for mm in masks[1:]:
                    m = jnp.logical_and(m, mm)
                t = jnp.where(m, t, 0.0)
            taps[(dh, dw)] = t

    b = b_ref[...]
    ssum = jnp.zeros_like(b)
    ssq = jnp.zeros_like(b)
    p = 0
    for rh in (0, 1):
        ch = 1 - rh
        for rw in (0, 1):
            cw = 1 - rw
            acc = None
            for mh in (0, 1):
                for mw in (0, 1):
                    d = jnp.dot(w_ref[p, mh * 2 + mw],
                                taps[(ch - mh, cw - mw)],
                                preferred_element_type=jnp.float32)
                    acc = d if acc is None else acc + d
            y = acc + b
            o_ref[0, p] = y
            ssum = ssum + jnp.sum(y, axis=1, keepdims=True)
            ssq = ssq + jnp.sum(y * y, axis=1, keepdims=True)
            p += 1
    sum_ref[0] = ssum
    ssq_ref[0] = ssq


def _bn_relu_interleave_kernel(c_ref, sum_ref, ssq_ref, g_ref, be_ref,
                               o_ref, *, H, W, Mtot, eps):
    HW = H * W
    mean = jnp.sum(sum_ref[...], axis=0) / Mtot
    var = jnp.maximum(jnp.sum(ssq_ref[...], axis=0) / Mtot - mean * mean, 0.0)
    ginv = g_ref[...] * jax.lax.rsqrt(var + eps)
    sc = ginv
    sh = be_ref[...] - mean * ginv
    y = jnp.maximum(c_ref[0] * sc + sh, 0.0)
    Cout = y.shape[1]
    g = jax.lax.broadcasted_iota(jnp.int32, (Cout, H, 4 * W), 2)
    half = g // (2 * W)
    gg = g % (2 * W)
    idx = (2 * (1 - half) + (1 - gg % 2)) * W + gg // 2
    for ii in (0, 1):
        yi = y[:, :, ii * HW:(ii + 1) * HW]
        src = jnp.concatenate(
            [yi[p].reshape(Cout, H, W) for p in range(4)], axis=-1)
        o_ref[ii] = jnp.take_along_axis(src, idx, axis=-1)


def _upsample(x, w_t, bias, gamma, beta, *, eps=1e-5):
    N, Cin, H, W = map(int, x.shape)
    _, Cout, K, _ = map(int, w_t.shape)
    assert K == 4 and N % 2 == 0
    HW = H * W
    P = 4
    NP = N // 2

    xp = x.reshape(NP, 2, Cin, HW)

    wms = (w_t.transpose(2, 3, 1, 0)
           .reshape(2, 2, 2, 2, Cout, Cin)
           .transpose(1, 3, 0, 2, 4, 5)
           .reshape(P, 4, Cout, Cin).astype(jnp.float32))
    b2 = bias.reshape(Cout, 1).astype(jnp.float32)

    conv, sums, ssq = pl.pallas_call(
        partial(_conv_stats_kernel, H=H, W=W),
        out_shape=(
            jax.ShapeDtypeStruct((NP, P, Cout, 2 * HW), jnp.float32),
            jax.ShapeDtypeStruct((NP, Cout, 1), jnp.float32),
            jax.ShapeDtypeStruct((NP, Cout, 1), jnp.float32),
        ),
        grid=(NP,),
        in_specs=[
            pl.BlockSpec((1, 2, Cin, HW), lambda n: (n, 0, 0, 0)),
            pl.BlockSpec((P, 4, Cout, Cin), lambda n: (0, 0, 0, 0)),
            pl.BlockSpec((Cout, 1), lambda n: (0, 0)),
        ],
        out_specs=(
            pl.BlockSpec((1, P, Cout, 2 * HW), lambda n: (n, 0, 0, 0)),
            pl.BlockSpec((1, Cout, 1), lambda n: (n, 0, 0)),
            pl.BlockSpec((1, Cout, 1), lambda n: (n, 0, 0)),
        ),
        compiler_params=pltpu.CompilerParams(
            dimension_semantics=("parallel",)),
    )(xp, wms, b2)

    g2 = gamma.reshape(Cout, 1).astype(jnp.float32)
    be2 = beta.reshape(Cout, 1).astype(jnp.float32)

    out = pl.pallas_call(
        partial(_bn_relu_interleave_kernel, H=H, W=W,
                Mtot=float(N * P * HW), eps=eps),
        out_shape=jax.ShapeDtypeStruct((N, Cout, H, 4 * W), jnp.float32),
        grid=(NP,),
        in_specs=[
            pl.BlockSpec((1, P, Cout, 2 * HW), lambda n: (n, 0, 0, 0)),
            pl.BlockSpec((NP, Cout, 1), lambda n: (0, 0, 0)),
            pl.BlockSpec((NP, Cout, 1), lambda n: (0, 0, 0)),
            pl.BlockSpec((Cout, 1), lambda n: (0, 0)),
            pl.BlockSpec((Cout, 1), lambda n: (0, 0)),
        ],
        out_specs=pl.BlockSpec((2, Cout, H, 4 * W), lambda n: (n, 0, 0, 0)),
        compiler_params=pltpu.CompilerParams(
            dimension_semantics=("parallel",)),
    )(conv, sums, ssq, g2, be2)

    return out.reshape(N, Cout, 2 * H, 2 * W)


def kernel(x, w_t, bias, gamma, beta):
    return _upsample(x, w_t, bias, gamma, beta)
